# Initial kernel scaffold; baseline (speedup 1.0000x reference)
import jax
import jax.numpy as jnp
from jax.experimental import pallas as pl


def kernel(x, y, anchors):
    raise NotImplementedError("write your pallas kernel here")


# speedup vs baseline: 18.2979x; 18.2979x over previous
import jax
import jax.numpy as jnp
from jax.experimental import pallas as pl
from jax.experimental.pallas import tpu as pltpu

N_BOXES = 20000
N_PAD = 20480
ROWS = N_PAD // 128
TOP_K = 100
IOU_THR = 0.3
INV_DET_LEN = 1.0 / 128.0


def _nms_body(scores_ref, y0_ref, y1_ref, y2_ref, y3_ref, y4_ref, y5_ref,
              y8_ref, y9_ref, ax_ref, ay_ref, out_ref, live_ref):
    f32 = jnp.float32
    scores = scores_ref[...]
    ax = ax_ref[...]
    ay = ay_ref[...]
    inv = f32(INV_DET_LEN)

    cx = y0_ref[...] * inv + ax
    cy = y1_ref[...] * inv + ay
    w = y2_ref[...] * inv
    h = y3_ref[...] * inv
    k0x = y4_ref[...] * inv + ax
    k0y = y5_ref[...] * inv + ay
    k2x = y8_ref[...] * inv + ax
    k2y = y9_ref[...] * inv + ay

    wh = w * f32(0.5)
    hh = h * f32(0.5)
    x1 = cx - wh
    x2 = cx + wh
    yl = cy - hh
    yh = cy + hh
    areas = (x2 - x1) * (yh - yl)

    row_i = jax.lax.broadcasted_iota(jnp.int32, (ROWS, 128), 0)
    lane_i = jax.lax.broadcasted_iota(jnp.int32, (ROWS, 128), 1)
    idx_arr = row_i * 128 + lane_i
    valid0 = idx_arr < N_BOXES

    big_i = jnp.int32(2**30)
    neg1 = f32(-1.0)

    m0 = jnp.max(jnp.where(valid0, scores, neg1))
    top0 = jnp.min(jnp.where(jnp.where(valid0, scores, neg1) == m0,
                             idx_arr, big_i))

    out_lane = jax.lax.broadcasted_iota(jnp.int32, (8, 128), 1)
    out_row = jax.lax.broadcasted_iota(jnp.int32, (8, 128), 0)

    live_ref[...] = jnp.where(valid0, scores, neg1)
    out_ref[...] = jnp.zeros((8, 128), dtype=f32)

    def body(i, carry):
        masked = live_ref[...]
        m = jnp.max(masked)
        idx = jnp.min(jnp.where(masked == m, idx_arr, big_i))
        idx = jnp.where(m == neg1, top0, idx)
        sel = idx_arr == idx

        neg_big = f32(-3e38)
        px1 = jnp.max(jnp.where(sel, x1, neg_big))
        pyl = jnp.max(jnp.where(sel, yl, neg_big))
        px2 = jnp.max(jnp.where(sel, x2, neg_big))
        pyh = jnp.max(jnp.where(sel, yh, neg_big))
        pk0x = jnp.max(jnp.where(sel, k0x, neg_big))
        pk0y = jnp.max(jnp.where(sel, k0y, neg_big))
        pk2x = jnp.max(jnp.where(sel, k2x, neg_big))
        pk2y = jnp.max(jnp.where(sel, k2y, neg_big))
        ps = jnp.max(jnp.where(sel, scores, neg_big))
        parea = (px2 - px1) * (pyh - pyl)

        xx1 = jnp.maximum(px1, x1)
        yy1 = jnp.maximum(pyl, yl)
        xx2 = jnp.minimum(px2, x2)
        yy2 = jnp.minimum(pyh, yh)
        iw = jnp.maximum(xx2 - xx1, f32(0.0))
        ih = jnp.maximum(yy2 - yy1, f32(0.0))
        inter = iw * ih
        iou = inter / (parea + areas - inter)
        keep_live = (iou <= f32(IOU_THR)) & jnp.logical_not(sel)
        live_ref[...] = jnp.where(keep_live, masked, neg1)

        pcx = (px1 + px2) * f32(0.5)
        pcy = (pyl + pyh) * f32(0.5)
        pw = px2 - px1

        col = jnp.where(out_row == 0, ps,
              jnp.where(out_row == 1, pcx,
              jnp.where(out_row == 2, pcy,
              jnp.where(out_row == 3, pw,
              jnp.where(out_row == 4, pk0x,
              jnp.where(out_row == 5, pk0y,
              jnp.where(out_row == 6, pk2x, pk2y)))))))
        out_ref[...] = jnp.where(out_lane == i, col, out_ref[...])
        return carry

    jax.lax.fori_loop(0, TOP_K, body, jnp.int32(0))


def _pad2d(v):
    return jnp.pad(v, (0, N_PAD - N_BOXES)).reshape(ROWS, 128)


def kernel(x, y, anchors):
    scores = jax.nn.sigmoid(x[0, :, 0])
    y0 = y[0]
    ins = [
        _pad2d(scores),
        _pad2d(y0[:, 0]), _pad2d(y0[:, 1]), _pad2d(y0[:, 2]), _pad2d(y0[:, 3]),
        _pad2d(y0[:, 4]), _pad2d(y0[:, 5]), _pad2d(y0[:, 8]), _pad2d(y0[:, 9]),
        _pad2d(anchors[:, 0]), _pad2d(anchors[:, 1]),
    ]
    acc = pl.pallas_call(
        _nms_body,
        out_shape=jax.ShapeDtypeStruct((8, 128), jnp.float32),
        scratch_shapes=[pltpu.VMEM((ROWS, 128), jnp.float32)],
    )(*ins)
    return acc[:, :TOP_K].T
